# baseline (speedup 1.0000x reference)
import functools

import jax
import jax.numpy as jnp
from jax import lax
from jax.experimental import pallas as pl
from jax.experimental.pallas import tpu as pltpu
from jax.experimental.pallas import tpu_sc as plsc

D_MODEL = 64
WIDE = 2 * D_MODEL
NUM_CORES = 2
NUM_SUBCORES = 16
NUM_WORKERS = NUM_CORES * NUM_SUBCORES
CHUNK = 256
NBUF = 2


@functools.lru_cache(maxsize=None)
def _make_lookup(B: int):
    assert B % (NUM_WORKERS * CHUNK * NBUF) == 0
    b_per_w = B // NUM_WORKERS
    n_chunks = b_per_w // CHUNK
    n_rounds = n_chunks // NBUF
    mesh = plsc.VectorSubcoreMesh(
        core_axis_name="c", subcore_axis_name="s",
        num_cores=NUM_CORES, num_subcores=NUM_SUBCORES)

    @functools.partial(
        pl.kernel,
        out_type=jax.ShapeDtypeStruct((B, D_MODEL), jnp.float32),
        mesh=mesh,
        scratch_types=[
            pltpu.VMEM((b_per_w,), jnp.int32),
            pltpu.VMEM((NBUF, CHUNK, WIDE), jnp.float32),
        ] + [pltpu.SemaphoreType.DMA] * (2 * NBUF),
        compiler_params=pltpu.CompilerParams(use_tc_tiling_on_sc=False),
    )
    def lookup(table_hbm, idx_hbm, out_hbm, idx_v, rows_v, *sems):
        sem_g = sems[:NBUF]
        sem_o = sems[NBUF:]
        wid = lax.axis_index("s") * NUM_CORES + lax.axis_index("c")
        base = wid * b_per_w
        pltpu.sync_copy(idx_hbm.at[pl.ds(base, b_per_w)], idx_v)

        def gather_desc(i, b):
            return pltpu.make_async_copy(
                table_hbm.at[idx_v.at[pl.ds(i * CHUNK, CHUNK)]],
                rows_v.at[b], sem_g[b])

        def out_desc(i, b):
            return pltpu.make_async_copy(
                rows_v.at[b, :, pl.ds(0, D_MODEL)],
                out_hbm.at[pl.ds(base + i * CHUNK, CHUNK)],
                sem_o[b])

        for b in range(NBUF):
            gather_desc(b, b).start()

        def body(r, carry):
            for b in range(NBUF):
                i = r * NBUF + b
                gather_desc(i, b).wait()
                out_desc(i, b).start()
            for b in range(NBUF):
                i = r * NBUF + b
                out_desc(i, b).wait()
                gather_desc(i + NBUF, b).start()
            return carry

        lax.fori_loop(0, n_rounds - 1, body, 0)

        r_last = n_rounds - 1
        for b in range(NBUF):
            i = r_last * NBUF + b
            gather_desc(i, b).wait()
            out_desc(i, b).start()
        for b in range(NBUF):
            out_desc(r_last * NBUF + b, b).wait()

    return lookup


@jax.jit
def kernel(x, weight):
    B, H = x.shape
    flat = jnp.transpose(x).reshape(B * H).astype(jnp.int32)
    wide = jnp.concatenate([weight, weight], axis=1)
    out_lin = _make_lookup(B * H)(wide, flat)
    return jnp.transpose(out_lin.reshape(H, B, D_MODEL), (1, 0, 2))


# speedup vs baseline: 1.5346x; 1.5346x over previous
import functools

import jax
import jax.numpy as jnp
from jax import lax
from jax.experimental import pallas as pl
from jax.experimental.pallas import tpu as pltpu
from jax.experimental.pallas import tpu_sc as plsc

D_MODEL = 64
NUM_CORES = 2
NUM_SUBCORES = 16
NUM_WORKERS = NUM_CORES * NUM_SUBCORES
CHUNK = 256
NBUF = 4

REPACK_COLS = 2048
REPACK_GRID = 245
HALF_ROWS = REPACK_COLS * REPACK_GRID
TABLE_ROWS = 2 * HALF_ROWS


def _repack_body(left_ref, right_ref, out_ref):
    out_ref[:, 0:D_MODEL] = jnp.transpose(left_ref[...], (1, 0))
    out_ref[:, D_MODEL:2 * D_MODEL] = jnp.transpose(right_ref[...], (1, 0))


@functools.lru_cache(maxsize=None)
def _make_repack(V: int):
    return pl.pallas_call(
        _repack_body,
        grid=(REPACK_GRID,),
        in_specs=[
            pl.BlockSpec((D_MODEL, REPACK_COLS), lambda i: (0, i)),
            pl.BlockSpec((D_MODEL, REPACK_COLS),
                         lambda i: (0, jnp.minimum(i + REPACK_GRID,
                                                   (V - 1) // REPACK_COLS))),
        ],
        out_specs=pl.BlockSpec((REPACK_COLS, 2 * D_MODEL), lambda i: (i, 0)),
        out_shape=jax.ShapeDtypeStruct((HALF_ROWS, 2 * D_MODEL), jnp.float32),
    )


@functools.lru_cache(maxsize=None)
def _make_lookup(B: int):
    assert B % (NUM_WORKERS * CHUNK * NBUF) == 0
    b_per_w = B // NUM_WORKERS
    n_chunks = b_per_w // CHUNK
    n_rounds = n_chunks // NBUF
    mesh = plsc.VectorSubcoreMesh(
        core_axis_name="c", subcore_axis_name="s",
        num_cores=NUM_CORES, num_subcores=NUM_SUBCORES)

    @functools.partial(
        pl.kernel,
        out_type=jax.ShapeDtypeStruct((B, D_MODEL), jnp.float32),
        mesh=mesh,
        scratch_types=[
            pltpu.VMEM((b_per_w,), jnp.int32),
            pltpu.VMEM((NBUF, CHUNK, D_MODEL), jnp.float32),
        ] + [pltpu.SemaphoreType.DMA] * (2 * NBUF),
        compiler_params=pltpu.CompilerParams(use_tc_tiling_on_sc=False),
    )
    def lookup(table_hbm, idx_hbm, out_hbm, idx_v, rows_v, *sems):
        sem_g = sems[:NBUF]
        sem_o = sems[NBUF:]
        wid = lax.axis_index("s") * NUM_CORES + lax.axis_index("c")
        base = wid * b_per_w
        pltpu.sync_copy(idx_hbm.at[pl.ds(base, b_per_w)], idx_v)

        def remap(j, carry):
            v = idx_v[pl.ds(j * 16, 16)]
            idx_v[pl.ds(j * 16, 16)] = jnp.where(
                v >= HALF_ROWS, 2 * v - (TABLE_ROWS - 1), 2 * v)
            return carry

        lax.fori_loop(0, b_per_w // 16, remap, 0)

        def gather_desc(i, b):
            return pltpu.make_async_copy(
                table_hbm.at[idx_v.at[pl.ds(i * CHUNK, CHUNK)]],
                rows_v.at[b], sem_g[b])

        def out_desc(i, b):
            return pltpu.make_async_copy(
                rows_v.at[b], out_hbm.at[pl.ds(base + i * CHUNK, CHUNK)],
                sem_o[b])

        for b in range(NBUF):
            gather_desc(b, b).start()

        def body(r, carry):
            for b in range(NBUF):
                i = r * NBUF + b
                gather_desc(i, b).wait()
                out_desc(i, b).start()
            for b in range(NBUF):
                i = r * NBUF + b
                out_desc(i, b).wait()
                gather_desc(i + NBUF, b).start()
            return carry

        lax.fori_loop(0, n_rounds - 1, body, 0)

        r_last = n_rounds - 1
        for b in range(NBUF):
            i = r_last * NBUF + b
            gather_desc(i, b).wait()
            out_desc(i, b).start()
        for b in range(NBUF):
            out_desc(r_last * NBUF + b, b).wait()

    return lookup


@jax.jit
def kernel(x, weight):
    B, H = x.shape
    V = weight.shape[0]
    flat = jnp.transpose(x).reshape(B * H).astype(jnp.int32)
    wt = jnp.transpose(weight)
    table = _make_repack(V)(wt, wt).reshape(TABLE_ROWS, D_MODEL)
    out_lin = _make_lookup(B * H)(table, flat)
    return jnp.transpose(out_lin.reshape(H, B, D_MODEL), (1, 0, 2))
